# baseline (speedup 1.0000x reference)
import functools

import jax
import jax.numpy as jnp
from jax import lax
from jax.experimental import pallas as pl
from jax.experimental.pallas import tpu as pltpu
from jax.experimental.pallas import tpu_sc as plsc

_TEMP_INV = 1.0 / 2.5
_K = 10
_ROWS = 64
_COLS = 32768
_LANES = 16
_GROUP = 16
_NGROUPS = _COLS // (_LANES * _GROUP)
_NWORK = 16
_ROWS_PER = _ROWS // _NWORK


def _scalar0(x):
    return lax.squeeze(lax.slice(x, (0,), (1,)), dimensions=(0,))


def _tree(x, lane, op):
    for sh in (8, 4, 2, 1):
        x = op(x, jnp.take(x, lane ^ sh))
    return x


def _bsort_asc(x, lane):
    for lk in (1, 2, 3, 4):
        for lj in range(lk - 1, -1, -1):
            j = 1 << lj
            p = jnp.take(x, lane ^ j)
            lo = jnp.minimum(x, p)
            hi = jnp.maximum(x, p)
            m = ((lane >> lk) ^ (lane >> lj)) & 1
            x = jnp.where(m == 0, lo, hi)
    return x


def _bmerge_asc(x, lane):
    for j in (8, 4, 2, 1):
        p = jnp.take(x, lane ^ j)
        lo = jnp.minimum(x, p)
        hi = jnp.maximum(x, p)
        x = jnp.where((lane & j) == 0, lo, hi)
    return x


def _colmax16(vs, lane):
    d = 1
    while len(vs) > 1:
        nvs = []
        for i in range(0, len(vs), 2):
            a, b = vs[i], vs[i + 1]
            ra = jnp.maximum(a, jnp.take(a, lane ^ d))
            rb = jnp.maximum(b, jnp.take(b, lane ^ d))
            nvs.append(jnp.where((lane & d) == 0, ra, rb))
        vs = nvs
        d *= 2
    return vs[0]


_mesh = plsc.VectorSubcoreMesh(core_axis_name="c", subcore_axis_name="s", num_cores=1)


@functools.partial(
    pl.kernel,
    mesh=_mesh,
    out_type=jax.ShapeDtypeStruct((_ROWS, _LANES), jnp.float32),
    scratch_types=[
        pltpu.VMEM((2, _COLS), jnp.float32),
        pltpu.VMEM((2 * _NGROUPS * _LANES,), jnp.float32),
        pltpu.VMEM((_LANES,), jnp.float32),
        pltpu.VMEM((_ROWS_PER, _LANES), jnp.float32),
    ],
)
def _sc_topk(s_hbm, out_hbm, rows_v, gmax_buf, top_ref, out_v):
    wid = lax.axis_index("s")

    lane = lax.iota(jnp.int32, _LANES)
    neg_inf_v = jnp.full((_LANES,), -jnp.inf, jnp.float32)

    _NCH = 4

    def p1_body(g, carry):
        out = []
        base = g * (_GROUP * _LANES)
        for r in range(2):
            gall = carry[r]
            gms = [None] * _NCH
            for jj in range(_GROUP):
                c = jj % _NCH
                v = rows_v[r, pl.ds(base + jj * _LANES, _LANES)]
                gms[c] = v if gms[c] is None else jnp.maximum(gms[c], v)
            gmax_g = jnp.maximum(jnp.maximum(gms[0], gms[1]),
                                 jnp.maximum(gms[2], gms[3]))
            gmax_buf[pl.ds(r * (_NGROUPS * _LANES) + g * _LANES, _LANES)] \
                = gmax_g
            out.append(jnp.maximum(gall, gmax_g))
        return tuple(out)

    for h in range(_ROWS_PER // 2):
      pltpu.sync_copy(s_hbm.at[pl.ds(wid * _ROWS_PER + h * 2, 2)], rows_v)
      galls = lax.fori_loop(0, _NGROUPS, p1_body, (neg_inf_v,) * 2)

      for r in range(2):
        gall = galls[r]

        gall_sorted = _bsort_asc(gall, lane)
        t0 = _scalar0(lax.slice(gall_sorted, (_LANES - _K,),
                                (_LANES - _K + 1,)))

        top_ref[...] = neg_inf_v
        lane_bit = jnp.left_shift(jnp.int32(1), lane)

        def merge_chunk(v):
            v_desc = lax.rev(_bsort_asc(v, lane), (0,))
            cand = jnp.maximum(top_ref[...], v_desc)
            top_ref[...] = _bmerge_asc(cand, lane)

        def bitmask_ge(vecs):
            cm = _colmax16(vecs, lane)
            bits = jnp.where(cm >= t0, lane_bit, jnp.int32(0))
            return _scalar0(_tree(bits, lane, jnp.bitwise_or))

        def p2_super(s, c):
            gbase = s * (_GROUP * _LANES * _LANES)
            gms = [gmax_buf[pl.ds(r * (_NGROUPS * _LANES)
                                  + s * (_LANES * _LANES) + j * _LANES,
                                  _LANES)] for j in range(_LANES)]
            gbm = bitmask_ge(gms)

            @pl.when(gbm != 0)
            def _():
                def g_body(j, cc):
                    @pl.when(((gbm >> j) & 1) != 0)
                    def _():
                        cbase = gbase + j * (_GROUP * _LANES)
                        vs = [rows_v[r, pl.ds(cbase + jj * _LANES, _LANES)]
                              for jj in range(_GROUP)]
                        cbm = bitmask_ge(vs)

                        def c_body(jj, ccc):
                            @pl.when(((cbm >> jj) & 1) != 0)
                            def _():
                                merge_chunk(
                                    rows_v[r, pl.ds(cbase + jj * _LANES,
                                                    _LANES)])
                            return ccc

                        lax.fori_loop(0, _GROUP, c_body, 0)
                    return cc

                lax.fori_loop(0, _LANES, g_body, 0)

            return c

        lax.fori_loop(0, _NGROUPS // _LANES, p2_super, 0)

        out_v[h * 2 + r, :] = top_ref[...]

    pltpu.sync_copy(out_v, out_hbm.at[pl.ds(wid * _ROWS_PER, _ROWS_PER)])


def _row_sums_body(x_ref, o_ref):
    x = x_ref[...]
    sig = 1.0 / (1.0 + jnp.exp(x * (-_TEMP_INV)))
    o_ref[...] = jnp.sum(sig, axis=1, keepdims=True)


def _loss_body(top_ref, sums_ref, o_ref):
    top = top_ref[...][:, _LANES - _K:]
    sig_top = 1.0 / (1.0 + jnp.exp(top * (-_TEMP_INV)))
    stk = jnp.sum(sig_top, axis=1, keepdims=True) / sums_ref[...]
    t = stk * jnp.log(stk + 1e-10)
    o_ref[...] = jnp.reshape(-jnp.sum(t) / _ROWS, (1, 1))


def kernel(S):
    sums = pl.pallas_call(
        _row_sums_body,
        out_shape=jax.ShapeDtypeStruct((_ROWS, 1), jnp.float32),
    )(S)
    top = _sc_topk(S)
    loss = pl.pallas_call(
        _loss_body,
        out_shape=jax.ShapeDtypeStruct((1, 1), jnp.float32),
    )(top, sums)
    return loss[0, 0]


# speedup vs baseline: 1.3690x; 1.3690x over previous
import functools

import jax
import jax.numpy as jnp
from jax import lax
from jax.experimental import pallas as pl
from jax.experimental.pallas import tpu as pltpu
from jax.experimental.pallas import tpu_sc as plsc

_TEMP_INV = 1.0 / 2.5
_K = 10
_ROWS = 64
_COLS = 32768
_LANES = 16
_GROUP = 16
_NGROUPS = _COLS // (_LANES * _GROUP)
_NWORK = 32
_ROWS_PER = _ROWS // _NWORK


def _scalar0(x):
    return lax.squeeze(lax.slice(x, (0,), (1,)), dimensions=(0,))


def _tree(x, lane, op):
    for sh in (8, 4, 2, 1):
        x = op(x, jnp.take(x, lane ^ sh))
    return x


def _bsort_asc(x, lane):
    for lk in (1, 2, 3, 4):
        for lj in range(lk - 1, -1, -1):
            j = 1 << lj
            p = jnp.take(x, lane ^ j)
            lo = jnp.minimum(x, p)
            hi = jnp.maximum(x, p)
            m = ((lane >> lk) ^ (lane >> lj)) & 1
            x = jnp.where(m == 0, lo, hi)
    return x


def _bmerge_asc(x, lane):
    for j in (8, 4, 2, 1):
        p = jnp.take(x, lane ^ j)
        lo = jnp.minimum(x, p)
        hi = jnp.maximum(x, p)
        x = jnp.where((lane & j) == 0, lo, hi)
    return x


def _colmax16(vs, lane):
    d = 1
    while len(vs) > 1:
        nvs = []
        for i in range(0, len(vs), 2):
            a, b = vs[i], vs[i + 1]
            ra = jnp.maximum(a, jnp.take(a, lane ^ d))
            rb = jnp.maximum(b, jnp.take(b, lane ^ d))
            nvs.append(jnp.where((lane & d) == 0, ra, rb))
        vs = nvs
        d *= 2
    return vs[0]


_mesh = plsc.VectorSubcoreMesh(core_axis_name="c", subcore_axis_name="s")


@functools.partial(
    pl.kernel,
    mesh=_mesh,
    out_type=jax.ShapeDtypeStruct((_ROWS, _LANES), jnp.float32),
    scratch_types=[
        pltpu.VMEM((_ROWS_PER, _COLS), jnp.float32),
        pltpu.VMEM((_ROWS_PER * _NGROUPS * _LANES,), jnp.float32),
        pltpu.VMEM((_LANES,), jnp.float32),
        pltpu.VMEM((_ROWS_PER, _LANES), jnp.float32),
        pltpu.SemaphoreType.DMA,
        pltpu.SemaphoreType.DMA,
    ],
)
def _sc_topk(s_hbm, out_hbm, rows_v, gmax_buf, top_ref, out_v, sem0, sem1):
    wid = lax.axis_index("s") * 2 + lax.axis_index("c")
    cps = [pltpu.make_async_copy(
        s_hbm.at[pl.ds(wid * _ROWS_PER + r, 1)],
        rows_v.at[pl.ds(r, 1)], sem)
        for r, sem in ((0, sem0), (1, sem1))]
    for cp in cps:
        cp.start()

    lane = lax.iota(jnp.int32, _LANES)
    neg_inf_v = jnp.full((_LANES,), -jnp.inf, jnp.float32)

    _NCH = 4

    def p1_row(r):
        def p1_body(g, gall):
            base = g * (_GROUP * _LANES)
            gms = [None] * _NCH
            for jj in range(_GROUP):
                c = jj % _NCH
                v = rows_v[r, pl.ds(base + jj * _LANES, _LANES)]
                gms[c] = v if gms[c] is None else jnp.maximum(gms[c], v)
            gmax_g = jnp.maximum(jnp.maximum(gms[0], gms[1]),
                                 jnp.maximum(gms[2], gms[3]))
            gmax_buf[pl.ds(r * (_NGROUPS * _LANES) + g * _LANES, _LANES)] \
                = gmax_g
            return jnp.maximum(gall, gmax_g)

        return lax.fori_loop(0, _NGROUPS, p1_body, neg_inf_v)

    for r in range(_ROWS_PER):
        cps[r].wait()
        gall = p1_row(r)

        gall_sorted = _bsort_asc(gall, lane)
        t0 = _scalar0(lax.slice(gall_sorted, (_LANES - _K,),
                                (_LANES - _K + 1,)))

        top_ref[...] = neg_inf_v
        lane_bit = jnp.left_shift(jnp.int32(1), lane)

        def merge_chunk(v):
            v_desc = lax.rev(_bsort_asc(v, lane), (0,))
            cand = jnp.maximum(top_ref[...], v_desc)
            top_ref[...] = _bmerge_asc(cand, lane)

        def bitmask_ge(vecs):
            cm = _colmax16(vecs, lane)
            bits = jnp.where(cm >= t0, lane_bit, jnp.int32(0))
            return _scalar0(_tree(bits, lane, jnp.bitwise_or))

        def p2_super(s, c):
            gbase = s * (_GROUP * _LANES * _LANES)
            gms = [gmax_buf[pl.ds(r * (_NGROUPS * _LANES)
                                  + s * (_LANES * _LANES) + j * _LANES,
                                  _LANES)] for j in range(_LANES)]
            gbm = bitmask_ge(gms)

            @pl.when(gbm != 0)
            def _():
                def g_body(j, cc):
                    @pl.when(((gbm >> j) & 1) != 0)
                    def _():
                        cbase = gbase + j * (_GROUP * _LANES)
                        vs = [rows_v[r, pl.ds(cbase + jj * _LANES, _LANES)]
                              for jj in range(_GROUP)]
                        cbm = bitmask_ge(vs)

                        def c_body(jj, ccc):
                            @pl.when(((cbm >> jj) & 1) != 0)
                            def _():
                                merge_chunk(
                                    rows_v[r, pl.ds(cbase + jj * _LANES,
                                                    _LANES)])
                            return ccc

                        lax.fori_loop(0, _GROUP, c_body, 0)
                    return cc

                lax.fori_loop(0, _LANES, g_body, 0)

            return c

        lax.fori_loop(0, _NGROUPS // _LANES, p2_super, 0)

        out_v[r, :] = top_ref[...]

    pltpu.sync_copy(out_v, out_hbm.at[pl.ds(wid * _ROWS_PER, _ROWS_PER)])


def _row_sums_body(x_ref, o_ref):
    x = x_ref[...]
    sig = 1.0 / (1.0 + jnp.exp(x * (-_TEMP_INV)))
    o_ref[...] = jnp.sum(sig, axis=1, keepdims=True)


def _loss_body(top_ref, sums_ref, o_ref):
    top = top_ref[...][:, _LANES - _K:]
    sig_top = 1.0 / (1.0 + jnp.exp(top * (-_TEMP_INV)))
    stk = jnp.sum(sig_top, axis=1, keepdims=True) / sums_ref[...]
    t = stk * jnp.log(stk + 1e-10)
    o_ref[...] = jnp.reshape(-jnp.sum(t) / _ROWS, (1, 1))


def kernel(S):
    sums = pl.pallas_call(
        _row_sums_body,
        out_shape=jax.ShapeDtypeStruct((_ROWS, 1), jnp.float32),
    )(S)
    top = _sc_topk(S)
    loss = pl.pallas_call(
        _loss_body,
        out_shape=jax.ShapeDtypeStruct((1, 1), jnp.float32),
    )(top, sums)
    return loss[0, 0]
